# baseline (speedup 1.0000x reference)
import functools

import jax
import jax.numpy as jnp
from jax import lax
from jax.experimental import pallas as pl
from jax.experimental.pallas import tpu as pltpu
from jax.experimental.pallas import tpu_sc as plsc

SEQ = 32768
DIM_HALF = 64

_info = plsc.get_sparse_core_info()
_NC, _NS = _info.num_cores, _info.num_subcores
_NW = _NC * _NS
_BPW = SEQ // _NW
_CHUNK = 256
_NCH = _BPW // _CHUNK


def _make_kernel():
  mesh = plsc.VectorSubcoreMesh(core_axis_name="c", subcore_axis_name="s")

  @functools.partial(
      pl.kernel,
      mesh=mesh,
      compiler_params=pltpu.CompilerParams(use_tc_tiling_on_sc=False),
      out_type=(
          jax.ShapeDtypeStruct((SEQ, DIM_HALF), jnp.float32),
          jax.ShapeDtypeStruct((SEQ, DIM_HALF), jnp.float32),
      ),
      scratch_types=[
          pltpu.VMEM((_BPW,), jnp.int32),
          pltpu.VMEM((_CHUNK, DIM_HALF), jnp.float32),
          pltpu.VMEM((_CHUNK, DIM_HALF), jnp.float32),
          pltpu.VMEM((_CHUNK, DIM_HALF), jnp.float32),
          pltpu.VMEM((_CHUNK, DIM_HALF), jnp.float32),
          pltpu.SemaphoreType.DMA,
          pltpu.SemaphoreType.DMA,
          pltpu.SemaphoreType.DMA,
          pltpu.SemaphoreType.DMA,
      ],
  )
  def rope_gather(pos_hbm, cos_hbm, sin_hbm, cos_out, sin_out,
                  idx_v, cos_b0, cos_b1, sin_b0, sin_b1,
                  gsem0, gsem1, wsem0, wsem1):
    wid = lax.axis_index("s") * _NC + lax.axis_index("c")
    base = wid * _BPW
    pltpu.sync_copy(pos_hbm.at[pl.ds(base, _BPW)], idx_v)

    cos_b = (cos_b0, cos_b1)
    sin_b = (sin_b0, sin_b1)
    gsem = (gsem0, gsem1)
    wsem = (wsem0, wsem1)

    def gather(c):
      p = c % 2
      idx_c = idx_v.at[pl.ds(c * _CHUNK, _CHUNK)]
      return (pltpu.async_copy(cos_hbm.at[idx_c], cos_b[p], gsem[p]),
              pltpu.async_copy(sin_hbm.at[idx_c], sin_b[p], gsem[p]))

    def write(c):
      p = c % 2
      off = base + c * _CHUNK
      return (pltpu.async_copy(cos_b[p], cos_out.at[pl.ds(off, _CHUNK)],
                               wsem[p]),
              pltpu.async_copy(sin_b[p], sin_out.at[pl.ds(off, _CHUNK)],
                               wsem[p]))

    pending_g = [None, None]
    pending_w = [None, None]
    pending_g[0] = gather(0)
    for c in range(_NCH):
      p = c % 2
      p1 = (c + 1) % 2
      if c + 1 < _NCH:
        if pending_w[p1] is not None:
          for cp in pending_w[p1]:
            cp.wait()
          pending_w[p1] = None
        pending_g[p1] = gather(c + 1)
      for cp in pending_g[p]:
        cp.wait()
      pending_w[p] = write(c)
    for p in range(2):
      if pending_w[p] is not None:
        for cp in pending_w[p]:
          cp.wait()

  return rope_gather


_rope_gather = _make_kernel()


@jax.jit
def kernel(positions, cos_cache, sin_cache):
  return _rope_gather(positions.astype(jnp.int32), cos_cache, sin_cache)
